# baseline (speedup 1.0000x reference)
import functools

import jax
import jax.numpy as jnp
from jax import lax
from jax.experimental import pallas as pl
from jax.experimental.pallas import tpu as pltpu

N = 10000
E = 160000
CHUNK = 128
EP = 163840
IDX_ROWS = EP // CHUNK
LEAK = 0.1
EPS = 1e-5
BE = 256
NBLK = EP // BE


def _lrelu(v):
    return jnp.where(v > 0, v, LEAK * v)


def _stats_body(v_ref, out_ref):
    blk = v_ref[...]
    s = jnp.sum(blk, axis=0, keepdims=True)
    sq = jnp.sum(blk * blk, axis=0, keepdims=True)
    acc = jnp.concatenate([s, sq], axis=0)

    @pl.when(pl.program_id(0) == 0)
    def _():
        out_ref[...] = acc

    @pl.when(pl.program_id(0) != 0)
    def _():
        out_ref[...] += acc


def _stats(v, bt):
    t, f = v.shape
    return pl.pallas_call(
        _stats_body,
        grid=(t // bt,),
        in_specs=[pl.BlockSpec((bt, f), lambda i: (i, 0))],
        out_specs=pl.BlockSpec((2, f), lambda i: (0, 0)),
        out_shape=jax.ShapeDtypeStruct((2, f), jnp.float32),
        compiler_params=pltpu.CompilerParams(
            dimension_semantics=("arbitrary",)),
    )(v)


def _node0_body(x_ref, sc_ref, sh_ref, root_ref, bias_ref, xbn_ref, x1r_ref):
    xbn = x_ref[...] * sc_ref[...] + sh_ref[...]
    xbn_ref[...] = xbn
    x1r_ref[...] = (
        jnp.dot(xbn, root_ref[...], preferred_element_type=jnp.float32)
        + bias_ref[...])


def _node0(x, scale, shift, root1, bias1):
    bn = 1000
    return pl.pallas_call(
        _node0_body,
        grid=(N // bn,),
        in_specs=[
            pl.BlockSpec((bn, 16), lambda i: (i, 0)),
            pl.BlockSpec((1, 16), lambda i: (0, 0)),
            pl.BlockSpec((1, 16), lambda i: (0, 0)),
            pl.BlockSpec((16, 32), lambda i: (0, 0)),
            pl.BlockSpec((1, 32), lambda i: (0, 0)),
        ],
        out_specs=[
            pl.BlockSpec((bn, 16), lambda i: (i, 0)),
            pl.BlockSpec((bn, 32), lambda i: (i, 0)),
        ],
        out_shape=[
            jax.ShapeDtypeStruct((N, 16), jnp.float32),
            jax.ShapeDtypeStruct((N, 32), jnp.float32),
        ],
    )(x, scale, shift, root1, bias1)


def _edge_body(fi, fo, e_ref, xs_ref, esc_ref, esh_ref, w1_ref, b1_ref,
               w2_ref, b2_ref, out_ref):
    ebn = e_ref[...] * esc_ref[...] + esh_ref[...]
    h = _lrelu(jnp.dot(ebn, w1_ref[...], preferred_element_type=jnp.float32)
               + b1_ref[...])
    xs = xs_ref[...]
    msg = jnp.zeros((out_ref.shape[0], fo), jnp.float32)
    per = 128 // fo
    for j in range(fi // per):
        z = (jnp.dot(h, w2_ref[:, 128 * j:128 * (j + 1)],
                     preferred_element_type=jnp.float32)
             + b2_ref[:, 128 * j:128 * (j + 1)])
        w = _lrelu(z)
        for t in range(per):
            i = per * j + t
            msg = msg + xs[:, i:i + 1] * w[:, fo * t:fo * (t + 1)]
    row = (pl.program_id(0) * out_ref.shape[0]
           + lax.broadcasted_iota(jnp.int32, msg.shape, 0))
    out_ref[...] = jnp.where(row < E, msg, 0.0)


def _edge_conv(fi, fo, e_pad, xsrc, escale, eshift, w1, b1, w2, b2):
    body = functools.partial(_edge_body, fi, fo)
    return pl.pallas_call(
        body,
        grid=(NBLK,),
        in_specs=[
            pl.BlockSpec((BE, 10), lambda i: (i, 0)),
            pl.BlockSpec((BE, fi), lambda i: (i, 0)),
            pl.BlockSpec((1, 10), lambda i: (0, 0)),
            pl.BlockSpec((1, 10), lambda i: (0, 0)),
            pl.BlockSpec((10, fi), lambda i: (0, 0)),
            pl.BlockSpec((1, fi), lambda i: (0, 0)),
            pl.BlockSpec((fi, fi * fo), lambda i: (0, 0)),
            pl.BlockSpec((1, fi * fo), lambda i: (0, 0)),
        ],
        out_specs=pl.BlockSpec((BE, fo), lambda i: (i, 0)),
        out_shape=jax.ShapeDtypeStruct((EP, fo), jnp.float32),
    )(e_pad, xsrc, escale, eshift, w1, b1, w2, b2)


def _node_body(base_ref, parts_ref, root_ref, bias_ref, xn_ref, xr_ref):
    xn = base_ref[...] + parts_ref[0] + parts_ref[1]
    xn_ref[...] = xn
    xr_ref[...] = (
        jnp.dot(xn, root_ref[...], preferred_element_type=jnp.float32)
        + bias_ref[...])


def _node_update(base, parts, root, bias):
    bn = 1000
    d = base.shape[1]
    do = root.shape[1]
    return pl.pallas_call(
        _node_body,
        grid=(N // bn,),
        in_specs=[
            pl.BlockSpec((bn, d), lambda i: (i, 0)),
            pl.BlockSpec((2, bn, d), lambda i: (0, i, 0)),
            pl.BlockSpec((d, do), lambda i: (0, 0)),
            pl.BlockSpec((1, do), lambda i: (0, 0)),
        ],
        out_specs=[
            pl.BlockSpec((bn, d), lambda i: (i, 0)),
            pl.BlockSpec((bn, do), lambda i: (i, 0)),
        ],
        out_shape=[
            jax.ShapeDtypeStruct((N, d), jnp.float32),
            jax.ShapeDtypeStruct((N, do), jnp.float32),
        ],
    )(base, parts, root, bias)


def _node_final_body(base_ref, parts_ref, xn_ref):
    xn_ref[...] = base_ref[...] + parts_ref[0] + parts_ref[1]


def _node_final(base, parts):
    bn = 1000
    d = base.shape[1]
    return pl.pallas_call(
        _node_final_body,
        grid=(N // bn,),
        in_specs=[
            pl.BlockSpec((bn, d), lambda i: (i, 0)),
            pl.BlockSpec((2, bn, d), lambda i: (0, i, 0)),
        ],
        out_specs=pl.BlockSpec((bn, d), lambda i: (i, 0)),
        out_shape=jax.ShapeDtypeStruct((N, d), jnp.float32),
    )(base, parts)


def _emlp_body(xs_ref, xd_ref, e_ref, esc_ref, esh_ref,
               w1s_ref, w1d_ref, w1e_ref, b1_ref, w2_ref, b2_ref,
               w3_ref, b3_ref, w4_ref, b4_ref, w5_ref, b5_ref, out_ref):
    ebn = e_ref[...] * esc_ref[...] + esh_ref[...]
    h = _lrelu(
        jnp.dot(xs_ref[...], w1s_ref[...], preferred_element_type=jnp.float32)
        + jnp.dot(xd_ref[...], w1d_ref[...], preferred_element_type=jnp.float32)
        + jnp.dot(ebn, w1e_ref[...], preferred_element_type=jnp.float32)
        + b1_ref[...])
    h = _lrelu(jnp.dot(h, w2_ref[...], preferred_element_type=jnp.float32)
               + b2_ref[...])
    h = _lrelu(jnp.dot(h, w3_ref[...], preferred_element_type=jnp.float32)
               + b3_ref[...])
    h = _lrelu(jnp.dot(h, w4_ref[...], preferred_element_type=jnp.float32)
               + b4_ref[...])
    out_ref[...] = (jnp.dot(h, w5_ref[...], preferred_element_type=jnp.float32)
                    + b5_ref[...])


def _edge_mlp(xs, xd, e_pad, escale, eshift, w1s, w1d, w1e, b1,
              w2, b2, w3, b3, w4, b4, w5, b5):
    cst = lambda i: (0, 0)
    return pl.pallas_call(
        _emlp_body,
        grid=(NBLK,),
        in_specs=[
            pl.BlockSpec((BE, 64), lambda i: (i, 0)),
            pl.BlockSpec((BE, 64), lambda i: (i, 0)),
            pl.BlockSpec((BE, 10), lambda i: (i, 0)),
            pl.BlockSpec((1, 10), cst),
            pl.BlockSpec((1, 10), cst),
            pl.BlockSpec((64, 64), cst),
            pl.BlockSpec((64, 64), cst),
            pl.BlockSpec((10, 64), cst),
            pl.BlockSpec((1, 64), cst),
            pl.BlockSpec((64, 32), cst),
            pl.BlockSpec((1, 32), cst),
            pl.BlockSpec((32, 16), cst),
            pl.BlockSpec((1, 16), cst),
            pl.BlockSpec((16, 8), cst),
            pl.BlockSpec((1, 8), cst),
            pl.BlockSpec((8, 2), cst),
            pl.BlockSpec((1, 2), cst),
        ],
        out_specs=pl.BlockSpec((BE, 2), lambda i: (i, 0)),
        out_shape=jax.ShapeDtypeStruct((EP, 2), jnp.float32),
    )(xs, xd, e_pad, escale, eshift, w1s, w1d, w1e, b1,
      w2, b2, w3, b3, w4, b4, w5, b5)


def _gather(table, idx2d):
    return jnp.take(table, idx2d.reshape(-1), axis=0)


def _scatter_add(msg, idx2d):
    agg = jax.ops.segment_sum(msg, idx2d.reshape(-1), num_segments=N)
    return jnp.stack([agg, jnp.zeros_like(agg)])


def kernel(x, edge_index, e, xbatch, params):
    p = params
    src = edge_index[0]
    dst = edge_index[1]
    pad = jnp.zeros((EP - E,), jnp.int32)
    src2d = jnp.concatenate([src, pad]).reshape(IDX_ROWS, CHUNK)
    dst2d = jnp.concatenate([dst, pad]).reshape(IDX_ROWS, CHUNK)
    e_pad = jnp.pad(e, ((0, EP - E), (0, 0)))

    xs_ = _stats(x, 1000)
    xm = xs_[0:1] / N
    xv = xs_[1:2] / N - xm * xm
    xscale = p['bn_node_g'][None, :] * lax.rsqrt(xv + EPS)
    xshift = p['bn_node_b'][None, :] - xm * xscale
    es_ = _stats(e, 8000)
    em = es_[0:1] / E
    ev = es_[1:2] / E - em * em
    escale = p['bn_edge_g'][None, :] * lax.rsqrt(ev + EPS)
    eshift = p['bn_edge_b'][None, :] - em * escale

    xbn, x1r = _node0(x, xscale, xshift, p['root1'], p['bias1'][None, :])

    xsrc1 = _gather(xbn, src2d)
    msg1 = _edge_conv(16, 32, e_pad, xsrc1, escale, eshift,
                      p['nn1_W1'], p['nn1_b1'][None, :],
                      p['nn1_W2'], p['nn1_b2'][None, :])
    parts1 = _scatter_add(msg1, dst2d)
    x1, x2r = _node_update(x1r, parts1, p['root2'], p['bias2'][None, :])

    xsrc2 = _gather(x1, src2d)
    msg2 = _edge_conv(32, 64, e_pad, xsrc2, escale, eshift,
                      p['nn2_W1'], p['nn2_b1'][None, :],
                      p['nn2_W2'], p['nn2_b2'][None, :])
    parts2 = _scatter_add(msg2, dst2d)
    x2 = _node_final(x2r, parts2)

    xsrc3 = _gather(x2, src2d)
    xdst3 = _gather(x2, dst2d)
    eo = _edge_mlp(xsrc3, xdst3, e_pad, escale, eshift,
                   p['eW1'][:64], p['eW1'][64:128], p['eW1'][128:],
                   p['eb1'][None, :],
                   p['eW2'], p['eb2'][None, :], p['eW3'], p['eb3'][None, :],
                   p['eW4'], p['eb4'][None, :], p['eW5'], p['eb5'][None, :])
    return eo[:E]


# speedup vs baseline: 2.8863x; 2.8863x over previous
import functools

import jax
import jax.numpy as jnp
from jax import lax
from jax.experimental import pallas as pl
from jax.experimental.pallas import tpu as pltpu
from jax.experimental.pallas import tpu_sc as plsc

N = 10000
E = 160000
CHUNK = 128
EP = 163840
IDX_ROWS = EP // CHUNK
LEAK = 0.1
EPS = 1e-5
BE = 512
NBLK = EP // BE


def _lrelu(v):
    return jnp.maximum(v, LEAK * v)


def _stats_body(v_ref, out_ref):
    blk = v_ref[...]
    s = jnp.sum(blk, axis=0, keepdims=True)
    sq = jnp.sum(blk * blk, axis=0, keepdims=True)
    acc = jnp.concatenate([s, sq], axis=0)

    @pl.when(pl.program_id(0) == 0)
    def _():
        out_ref[...] = acc

    @pl.when(pl.program_id(0) != 0)
    def _():
        out_ref[...] += acc


def _stats(v, bt):
    t, f = v.shape
    return pl.pallas_call(
        _stats_body,
        grid=(t // bt,),
        in_specs=[pl.BlockSpec((bt, f), lambda i: (i, 0))],
        out_specs=pl.BlockSpec((2, f), lambda i: (0, 0)),
        out_shape=jax.ShapeDtypeStruct((2, f), jnp.float32),
        compiler_params=pltpu.CompilerParams(
            dimension_semantics=("arbitrary",)),
    )(v)


def _node0_body(x_ref, sc_ref, sh_ref, root_ref, bias_ref, xbn_ref, x1r_ref):
    xbn = x_ref[...] * sc_ref[...] + sh_ref[...]
    bn = xbn.shape[0]
    xbn_ref[...] = jnp.concatenate(
        [xbn, jnp.zeros((bn, 112), jnp.float32)], axis=1)
    x1r_ref[...] = (
        jnp.dot(xbn, root_ref[...], preferred_element_type=jnp.float32)
        + bias_ref[...])


def _node0(x, scale, shift, root1, bias1):
    bn = 1000
    return pl.pallas_call(
        _node0_body,
        grid=(N // bn,),
        in_specs=[
            pl.BlockSpec((bn, 16), lambda i: (i, 0)),
            pl.BlockSpec((1, 16), lambda i: (0, 0)),
            pl.BlockSpec((1, 16), lambda i: (0, 0)),
            pl.BlockSpec((16, 32), lambda i: (0, 0)),
            pl.BlockSpec((1, 32), lambda i: (0, 0)),
        ],
        out_specs=[
            pl.BlockSpec((bn, 128), lambda i: (i, 0)),
            pl.BlockSpec((bn, 32), lambda i: (i, 0)),
        ],
        out_shape=[
            jax.ShapeDtypeStruct((N, 128), jnp.float32),
            jax.ShapeDtypeStruct((N, 32), jnp.float32),
        ],
    )(x, scale, shift, root1, bias1)


def _edge_body(fi, fo, limit, e_ref, xs_ref, esc_ref, esh_ref, w1_ref, b1_ref,
               w2_ref, b2_ref, r_ref, out_ref):
    be = out_ref.shape[0]
    ebn = e_ref[...] * esc_ref[...] + esh_ref[...]
    h = _lrelu(jnp.dot(ebn, w1_ref[...], preferred_element_type=jnp.float32)
               + b1_ref[...])
    xs = xs_ref[:, :fi]
    acc = jnp.zeros((be, 128), jnp.float32)
    for j in range(fi * fo // 128):
        z = (jnp.dot(h, w2_ref[:, 128 * j:128 * (j + 1)],
                     preferred_element_type=jnp.float32)
             + b2_ref[:, 128 * j:128 * (j + 1)])
        xr = jnp.dot(xs, r_ref[:, 128 * j:128 * (j + 1)],
                     preferred_element_type=jnp.float32)
        acc = acc + xr * _lrelu(z)
    w = 128
    while w > fo:
        w //= 2
        acc = acc[:, :w] + acc[:, w:2 * w]
    msg = jnp.concatenate(
        [acc, jnp.zeros((be, 128 - fo), jnp.float32)], axis=1)
    if limit is not None:
        row = (pl.program_id(0) * be
               + lax.broadcasted_iota(jnp.int32, msg.shape, 0))
        msg = jnp.where(row < limit, msg, 0.0)
    out_ref[...] = msg


def _edge_conv(fi, fo, limit, e_pad, xsrc, escale, eshift, w1, b1, w2, b2):
    rows = e_pad.shape[0]
    body = functools.partial(_edge_body, fi, fo, limit)
    rep = jnp.repeat(jnp.eye(fi, dtype=jnp.float32), fo, axis=1)
    return pl.pallas_call(
        body,
        grid=(rows // BE,),
        in_specs=[
            pl.BlockSpec((BE, 10), lambda i: (i, 0)),
            pl.BlockSpec((BE, 128), lambda i: (i, 0)),
            pl.BlockSpec((1, 10), lambda i: (0, 0)),
            pl.BlockSpec((1, 10), lambda i: (0, 0)),
            pl.BlockSpec((10, fi), lambda i: (0, 0)),
            pl.BlockSpec((1, fi), lambda i: (0, 0)),
            pl.BlockSpec((fi, fi * fo), lambda i: (0, 0)),
            pl.BlockSpec((1, fi * fo), lambda i: (0, 0)),
            pl.BlockSpec((fi, fi * fo), lambda i: (0, 0)),
        ],
        out_specs=pl.BlockSpec((BE, 128), lambda i: (i, 0)),
        out_shape=jax.ShapeDtypeStruct((rows, 128), jnp.float32),
    )(e_pad, xsrc, escale, eshift, w1, b1, w2, b2, rep)


def _node_body(d, base_ref, pa_ref, pb_ref, root_ref, bias_ref, xn_ref,
               xr_ref):
    xn = (base_ref[...] + pa_ref[0, :, :d] + pa_ref[1, :, :d]
          + pb_ref[0, :, :d] + pb_ref[1, :, :d])
    bn = xn.shape[0]
    xn_ref[...] = jnp.concatenate(
        [xn, jnp.zeros((bn, 128 - d), jnp.float32)], axis=1)
    xr_ref[...] = (
        jnp.dot(xn, root_ref[...], preferred_element_type=jnp.float32)
        + bias_ref[...])


def _node_update(base, pa, pb, root, bias):
    bn = 1000
    d = base.shape[1]
    do = root.shape[1]
    return pl.pallas_call(
        functools.partial(_node_body, d),
        grid=(N // bn,),
        in_specs=[
            pl.BlockSpec((bn, d), lambda i: (i, 0)),
            pl.BlockSpec((2, bn, 128), lambda i: (0, i, 0)),
            pl.BlockSpec((2, bn, 128), lambda i: (0, i, 0)),
            pl.BlockSpec((d, do), lambda i: (0, 0)),
            pl.BlockSpec((1, do), lambda i: (0, 0)),
        ],
        out_specs=[
            pl.BlockSpec((bn, 128), lambda i: (i, 0)),
            pl.BlockSpec((bn, do), lambda i: (i, 0)),
        ],
        out_shape=[
            jax.ShapeDtypeStruct((N, 128), jnp.float32),
            jax.ShapeDtypeStruct((N, do), jnp.float32),
        ],
    )(base, pa, pb, root, bias)


def _node_final_body(d, base_ref, pa_ref, pb_ref, xn_ref):
    xn = (base_ref[...] + pa_ref[0, :, :d] + pa_ref[1, :, :d]
          + pb_ref[0, :, :d] + pb_ref[1, :, :d])
    bn = xn.shape[0]
    xn_ref[...] = jnp.concatenate(
        [xn, jnp.zeros((bn, 128 - d), jnp.float32)], axis=1)


def _node_final(base, pa, pb):
    bn = 1000
    d = base.shape[1]
    return pl.pallas_call(
        functools.partial(_node_final_body, d),
        grid=(N // bn,),
        in_specs=[
            pl.BlockSpec((bn, d), lambda i: (i, 0)),
            pl.BlockSpec((2, bn, 128), lambda i: (0, i, 0)),
            pl.BlockSpec((2, bn, 128), lambda i: (0, i, 0)),
        ],
        out_specs=pl.BlockSpec((bn, 128), lambda i: (i, 0)),
        out_shape=jax.ShapeDtypeStruct((N, 128), jnp.float32),
    )(base, pa, pb)


def _emlp_body(xs_ref, xd_ref, e_ref, esc_ref, esh_ref,
               w1s_ref, w1d_ref, w1e_ref, b1_ref, w2_ref, b2_ref,
               w3_ref, b3_ref, w4_ref, b4_ref, w5_ref, b5_ref, out_ref):
    ebn = e_ref[...] * esc_ref[...] + esh_ref[...]
    h = _lrelu(
        jnp.dot(xs_ref[:, :64], w1s_ref[...], preferred_element_type=jnp.float32)
        + jnp.dot(xd_ref[:, :64], w1d_ref[...], preferred_element_type=jnp.float32)
        + jnp.dot(ebn, w1e_ref[...], preferred_element_type=jnp.float32)
        + b1_ref[...])
    h = _lrelu(jnp.dot(h, w2_ref[...], preferred_element_type=jnp.float32)
               + b2_ref[...])
    h = _lrelu(jnp.dot(h, w3_ref[...], preferred_element_type=jnp.float32)
               + b3_ref[...])
    h = _lrelu(jnp.dot(h, w4_ref[...], preferred_element_type=jnp.float32)
               + b4_ref[...])
    out_ref[...] = (jnp.dot(h, w5_ref[...], preferred_element_type=jnp.float32)
                    + b5_ref[...])


def _edge_mlp(xs, xd, e_pad, escale, eshift, w1s, w1d, w1e, b1,
              w2, b2, w3, b3, w4, b4, w5, b5):
    rows = e_pad.shape[0]
    cst = lambda i: (0, 0)
    return pl.pallas_call(
        _emlp_body,
        grid=(rows // BE,),
        in_specs=[
            pl.BlockSpec((BE, 128), lambda i: (i, 0)),
            pl.BlockSpec((BE, 128), lambda i: (i, 0)),
            pl.BlockSpec((BE, 10), lambda i: (i, 0)),
            pl.BlockSpec((1, 10), cst),
            pl.BlockSpec((1, 10), cst),
            pl.BlockSpec((64, 64), cst),
            pl.BlockSpec((64, 64), cst),
            pl.BlockSpec((10, 64), cst),
            pl.BlockSpec((1, 64), cst),
            pl.BlockSpec((64, 32), cst),
            pl.BlockSpec((1, 32), cst),
            pl.BlockSpec((32, 16), cst),
            pl.BlockSpec((1, 16), cst),
            pl.BlockSpec((16, 8), cst),
            pl.BlockSpec((1, 8), cst),
            pl.BlockSpec((8, 2), cst),
            pl.BlockSpec((1, 2), cst),
        ],
        out_specs=pl.BlockSpec((BE, 2), lambda i: (i, 0)),
        out_shape=jax.ShapeDtypeStruct((rows, 2), jnp.float32),
    )(xs, xd, e_pad, escale, eshift, w1s, w1d, w1e, b1,
      w2, b2, w3, b3, w4, b4, w5, b5)


@functools.cache
def _sc_mesh():
    return plsc.VectorSubcoreMesh(core_axis_name="c", subcore_axis_name="s")
_RPT = IDX_ROWS // 32
_EPT = _RPT * CHUNK


_GG = 2


def _gather_kern(d, rpt, idx_hbm, tab_hbm, out_hbm,
                 idx_v, buf0, buf1, sg0, sg1, so0, so1):
    cid = lax.axis_index("c")
    sid = lax.axis_index("s")
    wid = cid * 16 + sid
    ngrp = rpt // _GG
    ept = rpt * CHUNK
    pltpu.sync_copy(idx_hbm.at[pl.ds(pl.multiple_of(wid * rpt, 8), rpt)],
                    idx_v)
    bufs = (buf0, buf1)
    gsems = (sg0, sg1)
    osems = (so0, so1)
    gcopies = [None, None]
    ocopies = [None, None]

    def fire(g):
        p = g % 2
        cs = []
        for b in range(_GG):
            cs.append(pltpu.async_copy(
                tab_hbm.at[idx_v.at[g * _GG + b]],
                bufs[p].at[pl.ds(b * CHUNK, CHUNK)], gsems[p]))
        gcopies[p] = cs

    fire(0)
    for g in range(ngrp):
        p = g % 2
        if g + 1 < ngrp:
            if ocopies[1 - p] is not None:
                ocopies[1 - p].wait()
            fire(g + 1)
        for c in gcopies[p]:
            c.wait()
        base = pl.multiple_of(wid * ept + g * _GG * CHUNK, 8)
        ocopies[p] = pltpu.async_copy(
            bufs[p], out_hbm.at[pl.ds(base, _GG * CHUNK)], osems[p])
    ocopies[0].wait()
    ocopies[1].wait()


def _gather(table, idx2d):
    d = table.shape[1]
    rows = idx2d.shape[0]
    rpt = rows // 32
    f = functools.partial(
        pl.kernel,
        out_type=jax.ShapeDtypeStruct((rows * CHUNK, d), jnp.float32),
        mesh=_sc_mesh(),
        scratch_types=[
            pltpu.VMEM((rpt, CHUNK), jnp.int32),
            pltpu.VMEM((_GG * CHUNK, d), jnp.float32),
            pltpu.VMEM((_GG * CHUNK, d), jnp.float32),
            pltpu.SemaphoreType.DMA,
            pltpu.SemaphoreType.DMA,
            pltpu.SemaphoreType.DMA,
            pltpu.SemaphoreType.DMA,
        ],
    )(functools.partial(_gather_kern, d, rpt))
    return f(idx2d, table)


def _scatter_kern(d, rpt, idx_hbm, msg_hbm, zero_hbm, out_hbm,
                  idx_v, m0, m1, acc_sh, sm0, sm1):
    cid = lax.axis_index("c")
    sid = lax.axis_index("s")
    wid = cid * 16 + sid
    ept = rpt * CHUNK

    @pl.when(sid == 0)
    def _():
        pltpu.sync_copy(zero_hbm, acc_sh)

    plsc.subcore_barrier()
    pltpu.sync_copy(idx_hbm.at[pl.ds(pl.multiple_of(wid * rpt, 8), rpt)],
                    idx_v)
    bufs = (m0, m1)
    sems = (sm0, sm1)
    loads = [None, None]

    def fire(c):
        p = c % 2
        base = pl.multiple_of(wid * ept + c * CHUNK, 8)
        loads[p] = pltpu.async_copy(
            msg_hbm.at[pl.ds(base, CHUNK)], bufs[p], sems[p])

    fire(0)
    for c in range(rpt):
        p = c % 2
        loads[p].wait()
        if c + 1 < rpt:
            fire(c + 1)
        pltpu.sync_copy(bufs[p], acc_sh.at[idx_v.at[c]], add=True)
    plsc.subcore_barrier()

    @pl.when(sid < 10)
    def _():
        r0 = pl.multiple_of(sid * 1000, 8)
        pltpu.sync_copy(acc_sh.at[pl.ds(r0, 1000)],
                        out_hbm.at[cid].at[pl.ds(r0, 1000)])


def _scatter_add(msg, idx2d):
    d = msg.shape[1]
    rpt = idx2d.shape[0] // 32
    f = functools.partial(
        pl.kernel,
        out_type=jax.ShapeDtypeStruct((2, N, d), jnp.float32),
        mesh=_sc_mesh(),
        scratch_types=[
            pltpu.VMEM((rpt, CHUNK), jnp.int32),
            pltpu.VMEM((CHUNK, d), jnp.float32),
            pltpu.VMEM((CHUNK, d), jnp.float32),
            pltpu.VMEM_SHARED((N, d), jnp.float32),
            pltpu.SemaphoreType.DMA,
            pltpu.SemaphoreType.DMA,
        ],
    )(functools.partial(_scatter_kern, d, rpt))
    return f(idx2d, msg, jnp.zeros((N, d), jnp.float32))


_ROWS_A = 768
_EPA = _ROWS_A * CHUNK


def kernel(x, edge_index, e, xbatch, params):
    p = params
    src = edge_index[0]
    dst = edge_index[1]
    pad = jnp.zeros((EP - E,), jnp.int32)
    src2d = jnp.concatenate([src, pad]).reshape(IDX_ROWS, CHUNK)
    dst2d = jnp.concatenate([dst, pad]).reshape(IDX_ROWS, CHUNK)
    e_pad = jnp.pad(e, ((0, EP - E), (0, 0)))
    srch = (src2d[:_ROWS_A], src2d[_ROWS_A:])
    dsth = (dst2d[:_ROWS_A], dst2d[_ROWS_A:])
    eh = (e_pad[:_EPA], e_pad[_EPA:])
    limits = (None, E - _EPA)

    xs_ = _stats(x, 1000)
    xm = xs_[0:1] / N
    xv = xs_[1:2] / N - xm * xm
    xscale = p['bn_node_g'][None, :] * lax.rsqrt(xv + EPS)
    xshift = p['bn_node_b'][None, :] - xm * xscale
    es_ = _stats(e, 8000)
    em = es_[0:1] / E
    ev = es_[1:2] / E - em * em
    escale = p['bn_edge_g'][None, :] * lax.rsqrt(ev + EPS)
    eshift = p['bn_edge_b'][None, :] - em * escale

    xbn, x1r = _node0(x, xscale, xshift, p['root1'], p['bias1'][None, :])

    parts1 = []
    for h in range(2):
        xsrc = _gather(xbn, srch[h])
        msg = _edge_conv(16, 32, limits[h], eh[h], xsrc, escale, eshift,
                         p['nn1_W1'], p['nn1_b1'][None, :],
                         p['nn1_W2'], p['nn1_b2'][None, :])
        parts1.append(_scatter_add(msg, dsth[h]))
    x1, x2r = _node_update(x1r, parts1[0], parts1[1],
                           p['root2'], p['bias2'][None, :])

    parts2 = []
    for h in range(2):
        xsrc = _gather(x1, srch[h])
        msg = _edge_conv(32, 64, limits[h], eh[h], xsrc, escale, eshift,
                         p['nn2_W1'], p['nn2_b1'][None, :],
                         p['nn2_W2'], p['nn2_b2'][None, :])
        parts2.append(_scatter_add(msg, dsth[h]))
    x2 = _node_final(x2r, parts2[0], parts2[1])

    eos = []
    for h in range(2):
        xsrc = _gather(x2, srch[h])
        xdst = _gather(x2, dsth[h])
        eos.append(_edge_mlp(
            xsrc, xdst, eh[h], escale, eshift,
            p['eW1'][:64], p['eW1'][64:128], p['eW1'][128:],
            p['eb1'][None, :],
            p['eW2'], p['eb2'][None, :], p['eW3'], p['eb3'][None, :],
            p['eW4'], p['eb4'][None, :], p['eW5'], p['eb5'][None, :]))
    return jnp.concatenate(eos)[:E]


# speedup vs baseline: 3.1527x; 1.0923x over previous
import functools

import jax
import jax.numpy as jnp
from jax import lax
from jax.experimental import pallas as pl
from jax.experimental.pallas import tpu as pltpu
from jax.experimental.pallas import tpu_sc as plsc

N = 10000
E = 160000
CHUNK = 128
EP = 163840
IDX_ROWS = EP // CHUNK
LEAK = 0.1
EPS = 1e-5
BE = 1024
NBLK = EP // BE


def _lrelu(v):
    return jnp.maximum(v, LEAK * v)


def _stats_body(v_ref, out_ref):
    blk = v_ref[...]
    s = jnp.sum(blk, axis=0, keepdims=True)
    sq = jnp.sum(blk * blk, axis=0, keepdims=True)
    acc = jnp.concatenate([s, sq], axis=0)

    @pl.when(pl.program_id(0) == 0)
    def _():
        out_ref[...] = acc

    @pl.when(pl.program_id(0) != 0)
    def _():
        out_ref[...] += acc


def _stats(v, bt):
    t, f = v.shape
    return pl.pallas_call(
        _stats_body,
        grid=(t // bt,),
        in_specs=[pl.BlockSpec((bt, f), lambda i: (i, 0))],
        out_specs=pl.BlockSpec((2, f), lambda i: (0, 0)),
        out_shape=jax.ShapeDtypeStruct((2, f), jnp.float32),
        compiler_params=pltpu.CompilerParams(
            dimension_semantics=("arbitrary",)),
    )(v)


def _node0_body(x_ref, sc_ref, sh_ref, root_ref, bias_ref, xbn_ref, x1r_ref):
    xbn = x_ref[...] * sc_ref[...] + sh_ref[...]
    bn = xbn.shape[0]
    xbn_ref[...] = jnp.concatenate(
        [xbn, jnp.zeros((bn, 112), jnp.float32)], axis=1)
    x1r_ref[...] = (
        jnp.dot(xbn, root_ref[...], preferred_element_type=jnp.float32)
        + bias_ref[...])


def _node0(x, scale, shift, root1, bias1):
    bn = 1000
    return pl.pallas_call(
        _node0_body,
        grid=(N // bn,),
        in_specs=[
            pl.BlockSpec((bn, 16), lambda i: (i, 0)),
            pl.BlockSpec((1, 16), lambda i: (0, 0)),
            pl.BlockSpec((1, 16), lambda i: (0, 0)),
            pl.BlockSpec((16, 32), lambda i: (0, 0)),
            pl.BlockSpec((1, 32), lambda i: (0, 0)),
        ],
        out_specs=[
            pl.BlockSpec((bn, 128), lambda i: (i, 0)),
            pl.BlockSpec((bn, 32), lambda i: (i, 0)),
        ],
        out_shape=[
            jax.ShapeDtypeStruct((N, 128), jnp.float32),
            jax.ShapeDtypeStruct((N, 32), jnp.float32),
        ],
    )(x, scale, shift, root1, bias1)


def _edge_body(fi, fo, limit, e_ref, xs_ref, esc_ref, esh_ref, w1_ref, b1_ref,
               w2_ref, b2_ref, r_ref, out_ref):
    be = out_ref.shape[0]
    ebn = e_ref[...] * esc_ref[...] + esh_ref[...]
    h = _lrelu(jnp.dot(ebn, w1_ref[...], preferred_element_type=jnp.float32)
               + b1_ref[...])
    xs = xs_ref[:, :fi]
    acc = jnp.zeros((be, 128), jnp.float32)
    for j in range(fi * fo // 128):
        z = (jnp.dot(h, w2_ref[:, 128 * j:128 * (j + 1)],
                     preferred_element_type=jnp.float32)
             + b2_ref[:, 128 * j:128 * (j + 1)])
        xr = jnp.dot(xs, r_ref[:, 128 * j:128 * (j + 1)],
                     preferred_element_type=jnp.float32)
        acc = acc + xr * _lrelu(z)
    w = 128
    while w > fo:
        w //= 2
        acc = acc[:, :w] + acc[:, w:2 * w]
    msg = jnp.concatenate(
        [acc, jnp.zeros((be, 128 - fo), jnp.float32)], axis=1)
    if limit is not None:
        row = (pl.program_id(0) * be
               + lax.broadcasted_iota(jnp.int32, msg.shape, 0))
        msg = jnp.where(row < limit, msg, 0.0)
    out_ref[...] = msg


def _edge_conv(fi, fo, limit, e_pad, xsrc, escale, eshift, w1, b1, w2, b2):
    rows = e_pad.shape[0]
    body = functools.partial(_edge_body, fi, fo, limit)
    rep = jnp.repeat(jnp.eye(fi, dtype=jnp.float32), fo, axis=1)
    return pl.pallas_call(
        body,
        grid=(rows // BE,),
        in_specs=[
            pl.BlockSpec((BE, 10), lambda i: (i, 0)),
            pl.BlockSpec((BE, 128), lambda i: (i, 0)),
            pl.BlockSpec((1, 10), lambda i: (0, 0)),
            pl.BlockSpec((1, 10), lambda i: (0, 0)),
            pl.BlockSpec((10, fi), lambda i: (0, 0)),
            pl.BlockSpec((1, fi), lambda i: (0, 0)),
            pl.BlockSpec((fi, fi * fo), lambda i: (0, 0)),
            pl.BlockSpec((1, fi * fo), lambda i: (0, 0)),
            pl.BlockSpec((fi, fi * fo), lambda i: (0, 0)),
        ],
        out_specs=pl.BlockSpec((BE, 128), lambda i: (i, 0)),
        out_shape=jax.ShapeDtypeStruct((rows, 128), jnp.float32),
    )(e_pad, xsrc, escale, eshift, w1, b1, w2, b2, rep)


def _node_body(d, base_ref, pa_ref, pb_ref, root_ref, bias_ref, xn_ref,
               xr_ref):
    xn = (base_ref[...] + pa_ref[0, :, :d] + pa_ref[1, :, :d]
          + pb_ref[0, :, :d] + pb_ref[1, :, :d])
    bn = xn.shape[0]
    xn_ref[...] = jnp.concatenate(
        [xn, jnp.zeros((bn, 128 - d), jnp.float32)], axis=1)
    xr_ref[...] = (
        jnp.dot(xn, root_ref[...], preferred_element_type=jnp.float32)
        + bias_ref[...])


def _node_update(base, pa, pb, root, bias):
    bn = 1000
    d = base.shape[1]
    do = root.shape[1]
    return pl.pallas_call(
        functools.partial(_node_body, d),
        grid=(N // bn,),
        in_specs=[
            pl.BlockSpec((bn, d), lambda i: (i, 0)),
            pl.BlockSpec((2, bn, 128), lambda i: (0, i, 0)),
            pl.BlockSpec((2, bn, 128), lambda i: (0, i, 0)),
            pl.BlockSpec((d, do), lambda i: (0, 0)),
            pl.BlockSpec((1, do), lambda i: (0, 0)),
        ],
        out_specs=[
            pl.BlockSpec((bn, 128), lambda i: (i, 0)),
            pl.BlockSpec((bn, do), lambda i: (i, 0)),
        ],
        out_shape=[
            jax.ShapeDtypeStruct((N, 128), jnp.float32),
            jax.ShapeDtypeStruct((N, do), jnp.float32),
        ],
    )(base, pa, pb, root, bias)


def _node_final_body(d, base_ref, pa_ref, pb_ref, xn_ref):
    xn = (base_ref[...] + pa_ref[0, :, :d] + pa_ref[1, :, :d]
          + pb_ref[0, :, :d] + pb_ref[1, :, :d])
    bn = xn.shape[0]
    xn_ref[...] = jnp.concatenate(
        [xn, jnp.zeros((bn, 128 - d), jnp.float32)], axis=1)


def _node_final(base, pa, pb):
    bn = 1000
    d = base.shape[1]
    return pl.pallas_call(
        functools.partial(_node_final_body, d),
        grid=(N // bn,),
        in_specs=[
            pl.BlockSpec((bn, d), lambda i: (i, 0)),
            pl.BlockSpec((2, bn, 128), lambda i: (0, i, 0)),
            pl.BlockSpec((2, bn, 128), lambda i: (0, i, 0)),
        ],
        out_specs=pl.BlockSpec((bn, 128), lambda i: (i, 0)),
        out_shape=jax.ShapeDtypeStruct((N, 128), jnp.float32),
    )(base, pa, pb)


def _emlp_body(xs_ref, xd_ref, e_ref, esc_ref, esh_ref,
               w1s_ref, w1d_ref, w1e_ref, b1_ref, w2_ref, b2_ref,
               w3_ref, b3_ref, w4_ref, b4_ref, w5_ref, b5_ref, out_ref):
    ebn = e_ref[...] * esc_ref[...] + esh_ref[...]
    h = _lrelu(
        jnp.dot(xs_ref[:, :64], w1s_ref[...], preferred_element_type=jnp.float32)
        + jnp.dot(xd_ref[:, :64], w1d_ref[...], preferred_element_type=jnp.float32)
        + jnp.dot(ebn, w1e_ref[...], preferred_element_type=jnp.float32)
        + b1_ref[...])
    h = _lrelu(jnp.dot(h, w2_ref[...], preferred_element_type=jnp.float32)
               + b2_ref[...])
    h = _lrelu(jnp.dot(h, w3_ref[...], preferred_element_type=jnp.float32)
               + b3_ref[...])
    h = _lrelu(jnp.dot(h, w4_ref[...], preferred_element_type=jnp.float32)
               + b4_ref[...])
    out_ref[...] = (jnp.dot(h, w5_ref[...], preferred_element_type=jnp.float32)
                    + b5_ref[...])


def _edge_mlp(xs, xd, e_pad, escale, eshift, w1s, w1d, w1e, b1,
              w2, b2, w3, b3, w4, b4, w5, b5):
    rows = e_pad.shape[0]
    cst = lambda i: (0, 0)
    return pl.pallas_call(
        _emlp_body,
        grid=(rows // BE,),
        in_specs=[
            pl.BlockSpec((BE, 128), lambda i: (i, 0)),
            pl.BlockSpec((BE, 128), lambda i: (i, 0)),
            pl.BlockSpec((BE, 10), lambda i: (i, 0)),
            pl.BlockSpec((1, 10), cst),
            pl.BlockSpec((1, 10), cst),
            pl.BlockSpec((64, 64), cst),
            pl.BlockSpec((64, 64), cst),
            pl.BlockSpec((10, 64), cst),
            pl.BlockSpec((1, 64), cst),
            pl.BlockSpec((64, 32), cst),
            pl.BlockSpec((1, 32), cst),
            pl.BlockSpec((32, 16), cst),
            pl.BlockSpec((1, 16), cst),
            pl.BlockSpec((16, 8), cst),
            pl.BlockSpec((1, 8), cst),
            pl.BlockSpec((8, 2), cst),
            pl.BlockSpec((1, 2), cst),
        ],
        out_specs=pl.BlockSpec((BE, 2), lambda i: (i, 0)),
        out_shape=jax.ShapeDtypeStruct((rows, 2), jnp.float32),
    )(xs, xd, e_pad, escale, eshift, w1s, w1d, w1e, b1,
      w2, b2, w3, b3, w4, b4, w5, b5)


@functools.cache
def _sc_mesh():
    return plsc.VectorSubcoreMesh(core_axis_name="c", subcore_axis_name="s")
_RPT = IDX_ROWS // 32
_EPT = _RPT * CHUNK


_GG = 2


def _gather_kern(d, rpt, idx_hbm, tab_hbm, out_hbm,
                 idx_v, buf0, buf1, sg0, sg1, so0, so1):
    cid = lax.axis_index("c")
    sid = lax.axis_index("s")
    wid = cid * 16 + sid
    ngrp = rpt // _GG
    ept = rpt * CHUNK
    pltpu.sync_copy(idx_hbm.at[pl.ds(pl.multiple_of(wid * rpt, 8), rpt)],
                    idx_v)
    bufs = (buf0, buf1)
    gsems = (sg0, sg1)
    osems = (so0, so1)
    gcopies = [None, None]
    ocopies = [None, None]

    def fire(g):
        p = g % 2
        cs = []
        for b in range(_GG):
            cs.append(pltpu.async_copy(
                tab_hbm.at[idx_v.at[g * _GG + b]],
                bufs[p].at[pl.ds(b * CHUNK, CHUNK)], gsems[p]))
        gcopies[p] = cs

    fire(0)
    for g in range(ngrp):
        p = g % 2
        if g + 1 < ngrp:
            if ocopies[1 - p] is not None:
                ocopies[1 - p].wait()
            fire(g + 1)
        for c in gcopies[p]:
            c.wait()
        base = pl.multiple_of(wid * ept + g * _GG * CHUNK, 8)
        ocopies[p] = pltpu.async_copy(
            bufs[p], out_hbm.at[pl.ds(base, _GG * CHUNK)], osems[p])
    ocopies[0].wait()
    ocopies[1].wait()


def _gather(table, idx2d):
    d = table.shape[1]
    rows = idx2d.shape[0]
    rpt = rows // 32
    f = functools.partial(
        pl.kernel,
        out_type=jax.ShapeDtypeStruct((rows * CHUNK, d), jnp.float32),
        mesh=_sc_mesh(),
        scratch_types=[
            pltpu.VMEM((rpt, CHUNK), jnp.int32),
            pltpu.VMEM((_GG * CHUNK, d), jnp.float32),
            pltpu.VMEM((_GG * CHUNK, d), jnp.float32),
            pltpu.SemaphoreType.DMA,
            pltpu.SemaphoreType.DMA,
            pltpu.SemaphoreType.DMA,
            pltpu.SemaphoreType.DMA,
        ],
    )(functools.partial(_gather_kern, d, rpt))
    return f(idx2d, table)


def _scatter_kern(d, rpt, idx_hbm, msg_hbm, zero_hbm, out_hbm,
                  idx_v, m0, m1, acc_sh, sm0, sm1):
    cid = lax.axis_index("c")
    sid = lax.axis_index("s")
    wid = cid * 16 + sid
    ept = rpt * CHUNK

    @pl.when(sid == 0)
    def _():
        pltpu.sync_copy(zero_hbm, acc_sh)

    plsc.subcore_barrier()
    pltpu.sync_copy(idx_hbm.at[pl.ds(pl.multiple_of(wid * rpt, 8), rpt)],
                    idx_v)
    bufs = (m0, m1)
    sems = (sm0, sm1)
    loads = [None, None]

    def fire(c):
        p = c % 2
        base = pl.multiple_of(wid * ept + c * CHUNK, 8)
        loads[p] = pltpu.async_copy(
            msg_hbm.at[pl.ds(base, CHUNK)], bufs[p], sems[p])

    fire(0)
    for c in range(rpt):
        p = c % 2
        loads[p].wait()
        if c + 1 < rpt:
            fire(c + 1)
        pltpu.sync_copy(bufs[p], acc_sh.at[idx_v.at[c]], add=True)
    plsc.subcore_barrier()

    @pl.when(sid < 10)
    def _():
        r0 = pl.multiple_of(sid * 1000, 8)
        pltpu.sync_copy(acc_sh.at[pl.ds(r0, 1000)],
                        out_hbm.at[cid].at[pl.ds(r0, 1000)])


def _scatter_add(msg, idx2d):
    d = msg.shape[1]
    rpt = idx2d.shape[0] // 32
    f = functools.partial(
        pl.kernel,
        out_type=jax.ShapeDtypeStruct((2, N, d), jnp.float32),
        mesh=_sc_mesh(),
        scratch_types=[
            pltpu.VMEM((rpt, CHUNK), jnp.int32),
            pltpu.VMEM((CHUNK, d), jnp.float32),
            pltpu.VMEM((CHUNK, d), jnp.float32),
            pltpu.VMEM_SHARED((N, d), jnp.float32),
            pltpu.SemaphoreType.DMA,
            pltpu.SemaphoreType.DMA,
        ],
    )(functools.partial(_scatter_kern, d, rpt))
    return f(idx2d, msg, jnp.zeros((N, d), jnp.float32))


_ROWS_A = 768
_EPA = _ROWS_A * CHUNK


def kernel(x, edge_index, e, xbatch, params):
    p = params
    src = edge_index[0]
    dst = edge_index[1]
    pad = jnp.zeros((EP - E,), jnp.int32)
    src2d = jnp.concatenate([src, pad]).reshape(IDX_ROWS, CHUNK)
    dst2d = jnp.concatenate([dst, pad]).reshape(IDX_ROWS, CHUNK)
    e_pad = jnp.pad(e, ((0, EP - E), (0, 0)))
    srch = (src2d[:_ROWS_A], src2d[_ROWS_A:])
    dsth = (dst2d[:_ROWS_A], dst2d[_ROWS_A:])
    eh = (e_pad[:_EPA], e_pad[_EPA:])
    limits = (None, E - _EPA)

    xs_ = _stats(x, 1000)
    xm = xs_[0:1] / N
    xv = xs_[1:2] / N - xm * xm
    xscale = p['bn_node_g'][None, :] * lax.rsqrt(xv + EPS)
    xshift = p['bn_node_b'][None, :] - xm * xscale
    es_ = _stats(e, 8000)
    em = es_[0:1] / E
    ev = es_[1:2] / E - em * em
    escale = p['bn_edge_g'][None, :] * lax.rsqrt(ev + EPS)
    eshift = p['bn_edge_b'][None, :] - em * escale

    xbn, x1r = _node0(x, xscale, xshift, p['root1'], p['bias1'][None, :])

    parts1 = []
    for h in range(2):
        xsrc = _gather(xbn, srch[h])
        msg = _edge_conv(16, 32, limits[h], eh[h], xsrc, escale, eshift,
                         p['nn1_W1'], p['nn1_b1'][None, :],
                         p['nn1_W2'], p['nn1_b2'][None, :])
        parts1.append(_scatter_add(msg, dsth[h]))
    x1, x2r = _node_update(x1r, parts1[0], parts1[1],
                           p['root2'], p['bias2'][None, :])

    parts2 = []
    for h in range(2):
        xsrc = _gather(x1, srch[h])
        msg = _edge_conv(32, 64, limits[h], eh[h], xsrc, escale, eshift,
                         p['nn2_W1'], p['nn2_b1'][None, :],
                         p['nn2_W2'], p['nn2_b2'][None, :])
        parts2.append(_scatter_add(msg, dsth[h]))
    x2 = _node_final(x2r, parts2[0], parts2[1])

    eos = []
    for h in range(2):
        xsrc = _gather(x2, srch[h])
        xdst = _gather(x2, dsth[h])
        eos.append(_edge_mlp(
            xsrc, xdst, eh[h], escale, eshift,
            p['eW1'][:64], p['eW1'][64:128], p['eW1'][128:],
            p['eb1'][None, :],
            p['eW2'], p['eb2'][None, :], p['eW3'], p['eb3'][None, :],
            p['eW4'], p['eb4'][None, :], p['eW5'], p['eb5'][None, :]))
    return jnp.concatenate(eos)[:E]


# speedup vs baseline: 3.3125x; 1.0507x over previous
import functools

import jax
import jax.numpy as jnp
from jax import lax
from jax.experimental import pallas as pl
from jax.experimental.pallas import tpu as pltpu
from jax.experimental.pallas import tpu_sc as plsc

N = 10000
E = 160000
CHUNK = 128
EP = 163840
IDX_ROWS = EP // CHUNK
LEAK = 0.1
EPS = 1e-5
BE = 2048
NBLK = EP // BE


def _lrelu(v):
    return jnp.maximum(v, LEAK * v)


def _stats_body(v_ref, out_ref):
    blk = v_ref[...]
    s = jnp.sum(blk, axis=0, keepdims=True)
    sq = jnp.sum(blk * blk, axis=0, keepdims=True)
    acc = jnp.concatenate([s, sq], axis=0)

    @pl.when(pl.program_id(0) == 0)
    def _():
        out_ref[...] = acc

    @pl.when(pl.program_id(0) != 0)
    def _():
        out_ref[...] += acc


def _stats(v, bt):
    t, f = v.shape
    return pl.pallas_call(
        _stats_body,
        grid=(t // bt,),
        in_specs=[pl.BlockSpec((bt, f), lambda i: (i, 0))],
        out_specs=pl.BlockSpec((2, f), lambda i: (0, 0)),
        out_shape=jax.ShapeDtypeStruct((2, f), jnp.float32),
        compiler_params=pltpu.CompilerParams(
            dimension_semantics=("arbitrary",)),
    )(v)


def _node0_body(x_ref, sc_ref, sh_ref, root_ref, bias_ref, xbn_ref, x1r_ref):
    xbn = x_ref[...] * sc_ref[...] + sh_ref[...]
    bn = xbn.shape[0]
    xbn_ref[...] = jnp.concatenate(
        [xbn, jnp.zeros((bn, 112), jnp.float32)], axis=1)
    x1r_ref[...] = (
        jnp.dot(xbn, root_ref[...], preferred_element_type=jnp.float32)
        + bias_ref[...])


def _node0(x, scale, shift, root1, bias1):
    bn = 1000
    return pl.pallas_call(
        _node0_body,
        grid=(N // bn,),
        in_specs=[
            pl.BlockSpec((bn, 16), lambda i: (i, 0)),
            pl.BlockSpec((1, 16), lambda i: (0, 0)),
            pl.BlockSpec((1, 16), lambda i: (0, 0)),
            pl.BlockSpec((16, 32), lambda i: (0, 0)),
            pl.BlockSpec((1, 32), lambda i: (0, 0)),
        ],
        out_specs=[
            pl.BlockSpec((bn, 128), lambda i: (i, 0)),
            pl.BlockSpec((bn, 32), lambda i: (i, 0)),
        ],
        out_shape=[
            jax.ShapeDtypeStruct((N, 128), jnp.float32),
            jax.ShapeDtypeStruct((N, 32), jnp.float32),
        ],
    )(x, scale, shift, root1, bias1)


def _edge_body(fi, fo, limit, e_ref, xs_ref, esc_ref, esh_ref, w1_ref, b1_ref,
               w2_ref, b2_ref, r_ref, out_ref):
    be = out_ref.shape[0]
    ebn = e_ref[...] * esc_ref[...] + esh_ref[...]
    h = _lrelu(jnp.dot(ebn, w1_ref[...], preferred_element_type=jnp.float32)
               + b1_ref[...])
    xs = xs_ref[:, :fi]
    acc = jnp.zeros((be, 128), jnp.float32)
    for j in range(fi * fo // 128):
        z = (jnp.dot(h, w2_ref[:, 128 * j:128 * (j + 1)],
                     preferred_element_type=jnp.float32)
             + b2_ref[:, 128 * j:128 * (j + 1)])
        xr = jnp.dot(xs, r_ref[:, 128 * j:128 * (j + 1)],
                     preferred_element_type=jnp.float32)
        acc = acc + xr * _lrelu(z)
    w = 128
    while w > fo:
        w //= 2
        acc = acc[:, :w] + acc[:, w:2 * w]
    msg = jnp.concatenate(
        [acc, jnp.zeros((be, 128 - fo), jnp.float32)], axis=1)
    if limit is not None:
        row = (pl.program_id(0) * be
               + lax.broadcasted_iota(jnp.int32, msg.shape, 0))
        msg = jnp.where(row < limit, msg, 0.0)
    out_ref[...] = msg


def _edge_conv(fi, fo, limit, e_pad, xsrc, escale, eshift, w1, b1, w2, b2):
    rows = e_pad.shape[0]
    body = functools.partial(_edge_body, fi, fo, limit)
    rep = jnp.repeat(jnp.eye(fi, dtype=jnp.float32), fo, axis=1)
    return pl.pallas_call(
        body,
        grid=(rows // BE,),
        in_specs=[
            pl.BlockSpec((BE, 10), lambda i: (i, 0)),
            pl.BlockSpec((BE, 128), lambda i: (i, 0)),
            pl.BlockSpec((1, 10), lambda i: (0, 0)),
            pl.BlockSpec((1, 10), lambda i: (0, 0)),
            pl.BlockSpec((10, fi), lambda i: (0, 0)),
            pl.BlockSpec((1, fi), lambda i: (0, 0)),
            pl.BlockSpec((fi, fi * fo), lambda i: (0, 0)),
            pl.BlockSpec((1, fi * fo), lambda i: (0, 0)),
            pl.BlockSpec((fi, fi * fo), lambda i: (0, 0)),
        ],
        out_specs=pl.BlockSpec((BE, 128), lambda i: (i, 0)),
        out_shape=jax.ShapeDtypeStruct((rows, 128), jnp.float32),
    )(e_pad, xsrc, escale, eshift, w1, b1, w2, b2, rep)


def _node_body(d, base_ref, pa_ref, pb_ref, root_ref, bias_ref, xn_ref,
               xr_ref):
    xn = (base_ref[...] + pa_ref[0, :, :d] + pa_ref[1, :, :d]
          + pb_ref[0, :, :d] + pb_ref[1, :, :d])
    bn = xn.shape[0]
    xn_ref[...] = jnp.concatenate(
        [xn, jnp.zeros((bn, 128 - d), jnp.float32)], axis=1)
    xr_ref[...] = (
        jnp.dot(xn, root_ref[...], preferred_element_type=jnp.float32)
        + bias_ref[...])


def _node_update(base, pa, pb, root, bias):
    bn = 1000
    d = base.shape[1]
    do = root.shape[1]
    return pl.pallas_call(
        functools.partial(_node_body, d),
        grid=(N // bn,),
        in_specs=[
            pl.BlockSpec((bn, d), lambda i: (i, 0)),
            pl.BlockSpec((2, bn, 128), lambda i: (0, i, 0)),
            pl.BlockSpec((2, bn, 128), lambda i: (0, i, 0)),
            pl.BlockSpec((d, do), lambda i: (0, 0)),
            pl.BlockSpec((1, do), lambda i: (0, 0)),
        ],
        out_specs=[
            pl.BlockSpec((bn, 128), lambda i: (i, 0)),
            pl.BlockSpec((bn, do), lambda i: (i, 0)),
        ],
        out_shape=[
            jax.ShapeDtypeStruct((N, 128), jnp.float32),
            jax.ShapeDtypeStruct((N, do), jnp.float32),
        ],
    )(base, pa, pb, root, bias)


def _node_final_body(d, base_ref, pa_ref, pb_ref, xn_ref):
    xn = (base_ref[...] + pa_ref[0, :, :d] + pa_ref[1, :, :d]
          + pb_ref[0, :, :d] + pb_ref[1, :, :d])
    bn = xn.shape[0]
    xn_ref[...] = jnp.concatenate(
        [xn, jnp.zeros((bn, 128 - d), jnp.float32)], axis=1)


def _node_final(base, pa, pb):
    bn = 1000
    d = base.shape[1]
    return pl.pallas_call(
        functools.partial(_node_final_body, d),
        grid=(N // bn,),
        in_specs=[
            pl.BlockSpec((bn, d), lambda i: (i, 0)),
            pl.BlockSpec((2, bn, 128), lambda i: (0, i, 0)),
            pl.BlockSpec((2, bn, 128), lambda i: (0, i, 0)),
        ],
        out_specs=pl.BlockSpec((bn, 128), lambda i: (i, 0)),
        out_shape=jax.ShapeDtypeStruct((N, 128), jnp.float32),
    )(base, pa, pb)


def _emlp_body(xs_ref, xd_ref, e_ref, esc_ref, esh_ref,
               w1s_ref, w1d_ref, w1e_ref, b1_ref, w2_ref, b2_ref,
               w3_ref, b3_ref, w4_ref, b4_ref, w5_ref, b5_ref, out_ref):
    ebn = e_ref[...] * esc_ref[...] + esh_ref[...]
    h = _lrelu(
        jnp.dot(xs_ref[:, :64], w1s_ref[...], preferred_element_type=jnp.float32)
        + jnp.dot(xd_ref[:, :64], w1d_ref[...], preferred_element_type=jnp.float32)
        + jnp.dot(ebn, w1e_ref[...], preferred_element_type=jnp.float32)
        + b1_ref[...])
    h = _lrelu(jnp.dot(h, w2_ref[...], preferred_element_type=jnp.float32)
               + b2_ref[...])
    h = _lrelu(jnp.dot(h, w3_ref[...], preferred_element_type=jnp.float32)
               + b3_ref[...])
    h = _lrelu(jnp.dot(h, w4_ref[...], preferred_element_type=jnp.float32)
               + b4_ref[...])
    out_ref[...] = (jnp.dot(h, w5_ref[...], preferred_element_type=jnp.float32)
                    + b5_ref[...])


def _edge_mlp(xs, xd, e_pad, escale, eshift, w1s, w1d, w1e, b1,
              w2, b2, w3, b3, w4, b4, w5, b5):
    rows = e_pad.shape[0]
    cst = lambda i: (0, 0)
    return pl.pallas_call(
        _emlp_body,
        grid=(rows // BE,),
        in_specs=[
            pl.BlockSpec((BE, 128), lambda i: (i, 0)),
            pl.BlockSpec((BE, 128), lambda i: (i, 0)),
            pl.BlockSpec((BE, 10), lambda i: (i, 0)),
            pl.BlockSpec((1, 10), cst),
            pl.BlockSpec((1, 10), cst),
            pl.BlockSpec((64, 64), cst),
            pl.BlockSpec((64, 64), cst),
            pl.BlockSpec((10, 64), cst),
            pl.BlockSpec((1, 64), cst),
            pl.BlockSpec((64, 32), cst),
            pl.BlockSpec((1, 32), cst),
            pl.BlockSpec((32, 16), cst),
            pl.BlockSpec((1, 16), cst),
            pl.BlockSpec((16, 8), cst),
            pl.BlockSpec((1, 8), cst),
            pl.BlockSpec((8, 2), cst),
            pl.BlockSpec((1, 2), cst),
        ],
        out_specs=pl.BlockSpec((BE, 2), lambda i: (i, 0)),
        out_shape=jax.ShapeDtypeStruct((rows, 2), jnp.float32),
    )(xs, xd, e_pad, escale, eshift, w1s, w1d, w1e, b1,
      w2, b2, w3, b3, w4, b4, w5, b5)


@functools.cache
def _sc_mesh():
    return plsc.VectorSubcoreMesh(core_axis_name="c", subcore_axis_name="s")
_RPT = IDX_ROWS // 32
_EPT = _RPT * CHUNK


_GG = 2


def _gather_kern(d, rpt, idx_hbm, tab_hbm, out_hbm,
                 idx_v, buf0, buf1, sg0, sg1, so0, so1):
    cid = lax.axis_index("c")
    sid = lax.axis_index("s")
    wid = cid * 16 + sid
    ngrp = rpt // _GG
    ept = rpt * CHUNK
    pltpu.sync_copy(idx_hbm.at[pl.ds(pl.multiple_of(wid * rpt, 8), rpt)],
                    idx_v)
    bufs = (buf0, buf1)
    gsems = (sg0, sg1)
    osems = (so0, so1)
    gcopies = [None, None]
    ocopies = [None, None]

    def fire(g):
        p = g % 2
        cs = []
        for b in range(_GG):
            cs.append(pltpu.async_copy(
                tab_hbm.at[idx_v.at[g * _GG + b]],
                bufs[p].at[pl.ds(b * CHUNK, CHUNK)], gsems[p]))
        gcopies[p] = cs

    fire(0)
    for g in range(ngrp):
        p = g % 2
        if g + 1 < ngrp:
            if ocopies[1 - p] is not None:
                ocopies[1 - p].wait()
            fire(g + 1)
        for c in gcopies[p]:
            c.wait()
        base = pl.multiple_of(wid * ept + g * _GG * CHUNK, 8)
        ocopies[p] = pltpu.async_copy(
            bufs[p], out_hbm.at[pl.ds(base, _GG * CHUNK)], osems[p])
    ocopies[0].wait()
    ocopies[1].wait()


def _gather(table, idx2d):
    d = table.shape[1]
    rows = idx2d.shape[0]
    rpt = rows // 32
    f = functools.partial(
        pl.kernel,
        out_type=jax.ShapeDtypeStruct((rows * CHUNK, d), jnp.float32),
        mesh=_sc_mesh(),
        scratch_types=[
            pltpu.VMEM((rpt, CHUNK), jnp.int32),
            pltpu.VMEM((_GG * CHUNK, d), jnp.float32),
            pltpu.VMEM((_GG * CHUNK, d), jnp.float32),
            pltpu.SemaphoreType.DMA,
            pltpu.SemaphoreType.DMA,
            pltpu.SemaphoreType.DMA,
            pltpu.SemaphoreType.DMA,
        ],
    )(functools.partial(_gather_kern, d, rpt))
    return f(idx2d, table)


def _scatter_kern(d, rpt, idx_hbm, msg_hbm, zero_hbm, out_hbm,
                  idx_v, m0, m1, acc_sh, sm0, sm1):
    cid = lax.axis_index("c")
    sid = lax.axis_index("s")
    wid = cid * 16 + sid
    ept = rpt * CHUNK

    @pl.when(sid == 0)
    def _():
        pltpu.sync_copy(zero_hbm, acc_sh)

    plsc.subcore_barrier()
    pltpu.sync_copy(idx_hbm.at[pl.ds(pl.multiple_of(wid * rpt, 8), rpt)],
                    idx_v)
    bufs = (m0, m1)
    sems = (sm0, sm1)
    loads = [None, None]

    def fire(c):
        p = c % 2
        base = pl.multiple_of(wid * ept + c * CHUNK, 8)
        loads[p] = pltpu.async_copy(
            msg_hbm.at[pl.ds(base, CHUNK)], bufs[p], sems[p])

    fire(0)
    for c in range(rpt):
        p = c % 2
        loads[p].wait()
        if c + 1 < rpt:
            fire(c + 1)
        pltpu.sync_copy(bufs[p], acc_sh.at[idx_v.at[c]], add=True)
    plsc.subcore_barrier()

    @pl.when(sid < 10)
    def _():
        r0 = pl.multiple_of(sid * 1000, 8)
        pltpu.sync_copy(acc_sh.at[pl.ds(r0, 1000)],
                        out_hbm.at[cid].at[pl.ds(r0, 1000)])


def _scatter_add(msg, idx2d):
    d = msg.shape[1]
    rpt = idx2d.shape[0] // 32
    f = functools.partial(
        pl.kernel,
        out_type=jax.ShapeDtypeStruct((2, N, d), jnp.float32),
        mesh=_sc_mesh(),
        scratch_types=[
            pltpu.VMEM((rpt, CHUNK), jnp.int32),
            pltpu.VMEM((CHUNK, d), jnp.float32),
            pltpu.VMEM((CHUNK, d), jnp.float32),
            pltpu.VMEM_SHARED((N, d), jnp.float32),
            pltpu.SemaphoreType.DMA,
            pltpu.SemaphoreType.DMA,
        ],
    )(functools.partial(_scatter_kern, d, rpt))
    return f(idx2d, msg, jnp.zeros((N, d), jnp.float32))


_ROWS_A = 768
_EPA = _ROWS_A * CHUNK


def kernel(x, edge_index, e, xbatch, params):
    p = params
    src = edge_index[0]
    dst = edge_index[1]
    pad = jnp.zeros((EP - E,), jnp.int32)
    src2d = jnp.concatenate([src, pad]).reshape(IDX_ROWS, CHUNK)
    dst2d = jnp.concatenate([dst, pad]).reshape(IDX_ROWS, CHUNK)
    e_pad = jnp.pad(e, ((0, EP - E), (0, 0)))
    srch = (src2d[:_ROWS_A], src2d[_ROWS_A:])
    dsth = (dst2d[:_ROWS_A], dst2d[_ROWS_A:])
    eh = (e_pad[:_EPA], e_pad[_EPA:])
    limits = (None, E - _EPA)

    xs_ = _stats(x, 1000)
    xm = xs_[0:1] / N
    xv = xs_[1:2] / N - xm * xm
    xscale = p['bn_node_g'][None, :] * lax.rsqrt(xv + EPS)
    xshift = p['bn_node_b'][None, :] - xm * xscale
    es_ = _stats(e, 8000)
    em = es_[0:1] / E
    ev = es_[1:2] / E - em * em
    escale = p['bn_edge_g'][None, :] * lax.rsqrt(ev + EPS)
    eshift = p['bn_edge_b'][None, :] - em * escale

    xbn, x1r = _node0(x, xscale, xshift, p['root1'], p['bias1'][None, :])

    parts1 = []
    for h in range(2):
        xsrc = _gather(xbn, srch[h])
        msg = _edge_conv(16, 32, limits[h], eh[h], xsrc, escale, eshift,
                         p['nn1_W1'], p['nn1_b1'][None, :],
                         p['nn1_W2'], p['nn1_b2'][None, :])
        parts1.append(_scatter_add(msg, dsth[h]))
    x1, x2r = _node_update(x1r, parts1[0], parts1[1],
                           p['root2'], p['bias2'][None, :])

    parts2 = []
    for h in range(2):
        xsrc = _gather(x1, srch[h])
        msg = _edge_conv(32, 64, limits[h], eh[h], xsrc, escale, eshift,
                         p['nn2_W1'], p['nn2_b1'][None, :],
                         p['nn2_W2'], p['nn2_b2'][None, :])
        parts2.append(_scatter_add(msg, dsth[h]))
    x2 = _node_final(x2r, parts2[0], parts2[1])

    eos = []
    for h in range(2):
        xsrc = _gather(x2, srch[h])
        xdst = _gather(x2, dsth[h])
        eos.append(_edge_mlp(
            xsrc, xdst, eh[h], escale, eshift,
            p['eW1'][:64], p['eW1'][64:128], p['eW1'][128:],
            p['eb1'][None, :],
            p['eW2'], p['eb2'][None, :], p['eW3'], p['eb3'][None, :],
            p['eW4'], p['eb4'][None, :], p['eW5'], p['eb5'][None, :]))
    return jnp.concatenate(eos)[:E]


# speedup vs baseline: 3.3800x; 1.0204x over previous
import functools

import jax
import jax.numpy as jnp
from jax import lax
from jax.experimental import pallas as pl
from jax.experimental.pallas import tpu as pltpu
from jax.experimental.pallas import tpu_sc as plsc

N = 10000
E = 160000
CHUNK = 128
EP = 163840
IDX_ROWS = EP // CHUNK
LEAK = 0.1
EPS = 1e-5
BE = 4096
NBLK = EP // BE


def _lrelu(v):
    return jnp.maximum(v, LEAK * v)


def _stats_body(v_ref, out_ref):
    blk = v_ref[...]
    s = jnp.sum(blk, axis=0, keepdims=True)
    sq = jnp.sum(blk * blk, axis=0, keepdims=True)
    acc = jnp.concatenate([s, sq], axis=0)

    @pl.when(pl.program_id(0) == 0)
    def _():
        out_ref[...] = acc

    @pl.when(pl.program_id(0) != 0)
    def _():
        out_ref[...] += acc


def _stats(v, bt):
    t, f = v.shape
    return pl.pallas_call(
        _stats_body,
        grid=(t // bt,),
        in_specs=[pl.BlockSpec((bt, f), lambda i: (i, 0))],
        out_specs=pl.BlockSpec((2, f), lambda i: (0, 0)),
        out_shape=jax.ShapeDtypeStruct((2, f), jnp.float32),
        compiler_params=pltpu.CompilerParams(
            dimension_semantics=("arbitrary",)),
    )(v)


def _node0_body(x_ref, sc_ref, sh_ref, root_ref, bias_ref, xbn_ref, x1r_ref):
    xbn = x_ref[...] * sc_ref[...] + sh_ref[...]
    bn = xbn.shape[0]
    xbn_ref[...] = jnp.concatenate(
        [xbn, jnp.zeros((bn, 112), jnp.float32)], axis=1)
    x1r_ref[...] = (
        jnp.dot(xbn, root_ref[...], preferred_element_type=jnp.float32)
        + bias_ref[...])


def _node0(x, scale, shift, root1, bias1):
    bn = 1000
    return pl.pallas_call(
        _node0_body,
        grid=(N // bn,),
        in_specs=[
            pl.BlockSpec((bn, 16), lambda i: (i, 0)),
            pl.BlockSpec((1, 16), lambda i: (0, 0)),
            pl.BlockSpec((1, 16), lambda i: (0, 0)),
            pl.BlockSpec((16, 32), lambda i: (0, 0)),
            pl.BlockSpec((1, 32), lambda i: (0, 0)),
        ],
        out_specs=[
            pl.BlockSpec((bn, 128), lambda i: (i, 0)),
            pl.BlockSpec((bn, 32), lambda i: (i, 0)),
        ],
        out_shape=[
            jax.ShapeDtypeStruct((N, 128), jnp.float32),
            jax.ShapeDtypeStruct((N, 32), jnp.float32),
        ],
    )(x, scale, shift, root1, bias1)


def _edge_body(fi, fo, limit, e_ref, xs_ref, esc_ref, esh_ref, w1_ref, b1_ref,
               w2_ref, b2_ref, r_ref, out_ref):
    be = out_ref.shape[0]
    ebn = e_ref[...] * esc_ref[...] + esh_ref[...]
    h = _lrelu(jnp.dot(ebn, w1_ref[...], preferred_element_type=jnp.float32)
               + b1_ref[...])
    xs = xs_ref[:, :fi]
    acc = jnp.zeros((be, 128), jnp.float32)
    for j in range(fi * fo // 128):
        z = (jnp.dot(h, w2_ref[:, 128 * j:128 * (j + 1)],
                     preferred_element_type=jnp.float32)
             + b2_ref[:, 128 * j:128 * (j + 1)])
        xr = jnp.dot(xs, r_ref[:, 128 * j:128 * (j + 1)],
                     preferred_element_type=jnp.float32)
        acc = acc + xr * _lrelu(z)
    w = 128
    while w > fo:
        w //= 2
        acc = acc[:, :w] + acc[:, w:2 * w]
    msg = jnp.concatenate(
        [acc, jnp.zeros((be, 128 - fo), jnp.float32)], axis=1)
    if limit is not None:
        row = (pl.program_id(0) * be
               + lax.broadcasted_iota(jnp.int32, msg.shape, 0))
        msg = jnp.where(row < limit, msg, 0.0)
    out_ref[...] = msg


def _edge_conv(fi, fo, limit, e_pad, xsrc, escale, eshift, w1, b1, w2, b2):
    rows = e_pad.shape[0]
    body = functools.partial(_edge_body, fi, fo, limit)
    rep = jnp.repeat(jnp.eye(fi, dtype=jnp.float32), fo, axis=1)
    return pl.pallas_call(
        body,
        grid=(rows // BE,),
        in_specs=[
            pl.BlockSpec((BE, 10), lambda i: (i, 0)),
            pl.BlockSpec((BE, 128), lambda i: (i, 0)),
            pl.BlockSpec((1, 10), lambda i: (0, 0)),
            pl.BlockSpec((1, 10), lambda i: (0, 0)),
            pl.BlockSpec((10, fi), lambda i: (0, 0)),
            pl.BlockSpec((1, fi), lambda i: (0, 0)),
            pl.BlockSpec((fi, fi * fo), lambda i: (0, 0)),
            pl.BlockSpec((1, fi * fo), lambda i: (0, 0)),
            pl.BlockSpec((fi, fi * fo), lambda i: (0, 0)),
        ],
        out_specs=pl.BlockSpec((BE, 128), lambda i: (i, 0)),
        out_shape=jax.ShapeDtypeStruct((rows, 128), jnp.float32),
    )(e_pad, xsrc, escale, eshift, w1, b1, w2, b2, rep)


def _node_body(d, base_ref, pa_ref, pb_ref, root_ref, bias_ref, xn_ref,
               xr_ref):
    xn = (base_ref[...] + pa_ref[0, :, :d] + pa_ref[1, :, :d]
          + pb_ref[0, :, :d] + pb_ref[1, :, :d])
    bn = xn.shape[0]
    xn_ref[...] = jnp.concatenate(
        [xn, jnp.zeros((bn, 128 - d), jnp.float32)], axis=1)
    xr_ref[...] = (
        jnp.dot(xn, root_ref[...], preferred_element_type=jnp.float32)
        + bias_ref[...])


def _node_update(base, pa, pb, root, bias):
    bn = 1000
    d = base.shape[1]
    do = root.shape[1]
    return pl.pallas_call(
        functools.partial(_node_body, d),
        grid=(N // bn,),
        in_specs=[
            pl.BlockSpec((bn, d), lambda i: (i, 0)),
            pl.BlockSpec((2, bn, 128), lambda i: (0, i, 0)),
            pl.BlockSpec((2, bn, 128), lambda i: (0, i, 0)),
            pl.BlockSpec((d, do), lambda i: (0, 0)),
            pl.BlockSpec((1, do), lambda i: (0, 0)),
        ],
        out_specs=[
            pl.BlockSpec((bn, 128), lambda i: (i, 0)),
            pl.BlockSpec((bn, do), lambda i: (i, 0)),
        ],
        out_shape=[
            jax.ShapeDtypeStruct((N, 128), jnp.float32),
            jax.ShapeDtypeStruct((N, do), jnp.float32),
        ],
    )(base, pa, pb, root, bias)


def _node_final_body(d, base_ref, pa_ref, pb_ref, xn_ref):
    xn = (base_ref[...] + pa_ref[0, :, :d] + pa_ref[1, :, :d]
          + pb_ref[0, :, :d] + pb_ref[1, :, :d])
    bn = xn.shape[0]
    xn_ref[...] = jnp.concatenate(
        [xn, jnp.zeros((bn, 128 - d), jnp.float32)], axis=1)


def _node_final(base, pa, pb):
    bn = 1000
    d = base.shape[1]
    return pl.pallas_call(
        functools.partial(_node_final_body, d),
        grid=(N // bn,),
        in_specs=[
            pl.BlockSpec((bn, d), lambda i: (i, 0)),
            pl.BlockSpec((2, bn, 128), lambda i: (0, i, 0)),
            pl.BlockSpec((2, bn, 128), lambda i: (0, i, 0)),
        ],
        out_specs=pl.BlockSpec((bn, 128), lambda i: (i, 0)),
        out_shape=jax.ShapeDtypeStruct((N, 128), jnp.float32),
    )(base, pa, pb)


def _emlp_body(xs_ref, xd_ref, e_ref, esc_ref, esh_ref,
               w1s_ref, w1d_ref, w1e_ref, b1_ref, w2_ref, b2_ref,
               w3_ref, b3_ref, w4_ref, b4_ref, w5_ref, b5_ref, out_ref):
    ebn = e_ref[...] * esc_ref[...] + esh_ref[...]
    h = _lrelu(
        jnp.dot(xs_ref[:, :64], w1s_ref[...], preferred_element_type=jnp.float32)
        + jnp.dot(xd_ref[:, :64], w1d_ref[...], preferred_element_type=jnp.float32)
        + jnp.dot(ebn, w1e_ref[...], preferred_element_type=jnp.float32)
        + b1_ref[...])
    h = _lrelu(jnp.dot(h, w2_ref[...], preferred_element_type=jnp.float32)
               + b2_ref[...])
    h = _lrelu(jnp.dot(h, w3_ref[...], preferred_element_type=jnp.float32)
               + b3_ref[...])
    h = _lrelu(jnp.dot(h, w4_ref[...], preferred_element_type=jnp.float32)
               + b4_ref[...])
    out_ref[...] = (jnp.dot(h, w5_ref[...], preferred_element_type=jnp.float32)
                    + b5_ref[...])


def _edge_mlp(xs, xd, e_pad, escale, eshift, w1s, w1d, w1e, b1,
              w2, b2, w3, b3, w4, b4, w5, b5):
    rows = e_pad.shape[0]
    cst = lambda i: (0, 0)
    return pl.pallas_call(
        _emlp_body,
        grid=(rows // BE,),
        in_specs=[
            pl.BlockSpec((BE, 128), lambda i: (i, 0)),
            pl.BlockSpec((BE, 128), lambda i: (i, 0)),
            pl.BlockSpec((BE, 10), lambda i: (i, 0)),
            pl.BlockSpec((1, 10), cst),
            pl.BlockSpec((1, 10), cst),
            pl.BlockSpec((64, 64), cst),
            pl.BlockSpec((64, 64), cst),
            pl.BlockSpec((10, 64), cst),
            pl.BlockSpec((1, 64), cst),
            pl.BlockSpec((64, 32), cst),
            pl.BlockSpec((1, 32), cst),
            pl.BlockSpec((32, 16), cst),
            pl.BlockSpec((1, 16), cst),
            pl.BlockSpec((16, 8), cst),
            pl.BlockSpec((1, 8), cst),
            pl.BlockSpec((8, 2), cst),
            pl.BlockSpec((1, 2), cst),
        ],
        out_specs=pl.BlockSpec((BE, 2), lambda i: (i, 0)),
        out_shape=jax.ShapeDtypeStruct((rows, 2), jnp.float32),
    )(xs, xd, e_pad, escale, eshift, w1s, w1d, w1e, b1,
      w2, b2, w3, b3, w4, b4, w5, b5)


@functools.cache
def _sc_mesh():
    return plsc.VectorSubcoreMesh(core_axis_name="c", subcore_axis_name="s")
_RPT = IDX_ROWS // 32
_EPT = _RPT * CHUNK


_GG = 2


def _gather_kern(d, rpt, idx_hbm, tab_hbm, out_hbm,
                 idx_v, buf0, buf1, sg0, sg1, so0, so1):
    cid = lax.axis_index("c")
    sid = lax.axis_index("s")
    wid = cid * 16 + sid
    ngrp = rpt // _GG
    ept = rpt * CHUNK
    pltpu.sync_copy(idx_hbm.at[pl.ds(pl.multiple_of(wid * rpt, 8), rpt)],
                    idx_v)
    bufs = (buf0, buf1)
    gsems = (sg0, sg1)
    osems = (so0, so1)
    gcopies = [None, None]
    ocopies = [None, None]

    def fire(g):
        p = g % 2
        cs = []
        for b in range(_GG):
            cs.append(pltpu.async_copy(
                tab_hbm.at[idx_v.at[g * _GG + b]],
                bufs[p].at[pl.ds(b * CHUNK, CHUNK)], gsems[p]))
        gcopies[p] = cs

    fire(0)
    for g in range(ngrp):
        p = g % 2
        if g + 1 < ngrp:
            if ocopies[1 - p] is not None:
                ocopies[1 - p].wait()
            fire(g + 1)
        for c in gcopies[p]:
            c.wait()
        base = pl.multiple_of(wid * ept + g * _GG * CHUNK, 8)
        ocopies[p] = pltpu.async_copy(
            bufs[p], out_hbm.at[pl.ds(base, _GG * CHUNK)], osems[p])
    ocopies[0].wait()
    ocopies[1].wait()


def _gather(table, idx2d):
    d = table.shape[1]
    rows = idx2d.shape[0]
    rpt = rows // 32
    f = functools.partial(
        pl.kernel,
        out_type=jax.ShapeDtypeStruct((rows * CHUNK, d), jnp.float32),
        mesh=_sc_mesh(),
        scratch_types=[
            pltpu.VMEM((rpt, CHUNK), jnp.int32),
            pltpu.VMEM((_GG * CHUNK, d), jnp.float32),
            pltpu.VMEM((_GG * CHUNK, d), jnp.float32),
            pltpu.SemaphoreType.DMA,
            pltpu.SemaphoreType.DMA,
            pltpu.SemaphoreType.DMA,
            pltpu.SemaphoreType.DMA,
        ],
    )(functools.partial(_gather_kern, d, rpt))
    return f(idx2d, table)


def _scatter_kern(d, rpt, idx_hbm, msg_hbm, zero_hbm, out_hbm,
                  idx_v, m0, m1, acc_sh, sm0, sm1):
    cid = lax.axis_index("c")
    sid = lax.axis_index("s")
    wid = cid * 16 + sid
    ept = rpt * CHUNK

    @pl.when(sid == 0)
    def _():
        pltpu.sync_copy(zero_hbm, acc_sh)

    plsc.subcore_barrier()
    pltpu.sync_copy(idx_hbm.at[pl.ds(pl.multiple_of(wid * rpt, 8), rpt)],
                    idx_v)
    bufs = (m0, m1)
    sems = (sm0, sm1)
    loads = [None, None]

    def fire(c):
        p = c % 2
        base = pl.multiple_of(wid * ept + c * CHUNK, 8)
        loads[p] = pltpu.async_copy(
            msg_hbm.at[pl.ds(base, CHUNK)], bufs[p], sems[p])

    fire(0)
    for c in range(rpt):
        p = c % 2
        loads[p].wait()
        if c + 1 < rpt:
            fire(c + 1)
        pltpu.sync_copy(bufs[p], acc_sh.at[idx_v.at[c]], add=True)
    plsc.subcore_barrier()

    @pl.when(sid < 10)
    def _():
        r0 = pl.multiple_of(sid * 1000, 8)
        pltpu.sync_copy(acc_sh.at[pl.ds(r0, 1000)],
                        out_hbm.at[cid].at[pl.ds(r0, 1000)])


def _scatter_add(msg, idx2d):
    d = msg.shape[1]
    rpt = idx2d.shape[0] // 32
    f = functools.partial(
        pl.kernel,
        out_type=jax.ShapeDtypeStruct((2, N, d), jnp.float32),
        mesh=_sc_mesh(),
        scratch_types=[
            pltpu.VMEM((rpt, CHUNK), jnp.int32),
            pltpu.VMEM((CHUNK, d), jnp.float32),
            pltpu.VMEM((CHUNK, d), jnp.float32),
            pltpu.VMEM_SHARED((N, d), jnp.float32),
            pltpu.SemaphoreType.DMA,
            pltpu.SemaphoreType.DMA,
        ],
    )(functools.partial(_scatter_kern, d, rpt))
    return f(idx2d, msg, jnp.zeros((N, d), jnp.float32))


_ROWS_A = 768
_EPA = _ROWS_A * CHUNK


def kernel(x, edge_index, e, xbatch, params):
    p = params
    src = edge_index[0]
    dst = edge_index[1]
    pad = jnp.zeros((EP - E,), jnp.int32)
    src2d = jnp.concatenate([src, pad]).reshape(IDX_ROWS, CHUNK)
    dst2d = jnp.concatenate([dst, pad]).reshape(IDX_ROWS, CHUNK)
    e_pad = jnp.pad(e, ((0, EP - E), (0, 0)))
    srch = (src2d[:_ROWS_A], src2d[_ROWS_A:])
    dsth = (dst2d[:_ROWS_A], dst2d[_ROWS_A:])
    eh = (e_pad[:_EPA], e_pad[_EPA:])
    limits = (None, E - _EPA)

    xs_ = _stats(x, 1000)
    xm = xs_[0:1] / N
    xv = xs_[1:2] / N - xm * xm
    xscale = p['bn_node_g'][None, :] * lax.rsqrt(xv + EPS)
    xshift = p['bn_node_b'][None, :] - xm * xscale
    es_ = _stats(e, 8000)
    em = es_[0:1] / E
    ev = es_[1:2] / E - em * em
    escale = p['bn_edge_g'][None, :] * lax.rsqrt(ev + EPS)
    eshift = p['bn_edge_b'][None, :] - em * escale

    xbn, x1r = _node0(x, xscale, xshift, p['root1'], p['bias1'][None, :])

    parts1 = []
    for h in range(2):
        xsrc = _gather(xbn, srch[h])
        msg = _edge_conv(16, 32, limits[h], eh[h], xsrc, escale, eshift,
                         p['nn1_W1'], p['nn1_b1'][None, :],
                         p['nn1_W2'], p['nn1_b2'][None, :])
        parts1.append(_scatter_add(msg, dsth[h]))
    x1, x2r = _node_update(x1r, parts1[0], parts1[1],
                           p['root2'], p['bias2'][None, :])

    parts2 = []
    for h in range(2):
        xsrc = _gather(x1, srch[h])
        msg = _edge_conv(32, 64, limits[h], eh[h], xsrc, escale, eshift,
                         p['nn2_W1'], p['nn2_b1'][None, :],
                         p['nn2_W2'], p['nn2_b2'][None, :])
        parts2.append(_scatter_add(msg, dsth[h]))
    x2 = _node_final(x2r, parts2[0], parts2[1])

    eos = []
    for h in range(2):
        xsrc = _gather(x2, srch[h])
        xdst = _gather(x2, dsth[h])
        eos.append(_edge_mlp(
            xsrc, xdst, eh[h], escale, eshift,
            p['eW1'][:64], p['eW1'][64:128], p['eW1'][128:],
            p['eb1'][None, :],
            p['eW2'], p['eb2'][None, :], p['eW3'], p['eb3'][None, :],
            p['eW4'], p['eb4'][None, :], p['eW5'], p['eb5'][None, :]))
    return jnp.concatenate(eos)[:E]
